# baseline (speedup 1.0000x reference)
import functools

import jax
import jax.numpy as jnp
from jax import lax
from jax.experimental import pallas as pl
from jax.experimental.pallas import tpu as pltpu
from jax.experimental.pallas import tpu_sc as plsc

N = 50000
K = 16
IN_CH = 128
OUT_CH = 256
D_F = 64
D_X = 16
PB_A = 2000
PB_C = 400
CB = 1000


def _lnorm(x, g, b, eps=1e-5):
    n_inv = 1.0 / x.shape[-1]
    mu = jnp.sum(x, axis=-1, keepdims=True) * n_inv
    var = jnp.sum(x * x, axis=-1, keepdims=True) * n_inv - mu * mu
    r = lax.rsqrt(var + eps)
    return (x - mu) * r * g + b


def _lrelu(x):
    return jnp.where(x >= 0, x, 0.1 * x)


def _ln_aug(pre, c, cw, eps=1e-5):
    n_inv = 1.0 / cw
    x = pre[:, :cw] + c[0:1]
    mu = pre[:, cw:cw + 1] * n_inv + c[3:4, 0:1]
    var = jnp.sum(x * x, axis=-1, keepdims=True) * n_inv - mu * mu
    r = lax.rsqrt(var + eps)
    return (x - mu) * r * c[1:2] + c[2:3]


def _stage_a_body(feats_ref, xyz_ref, u1w_ref, u1c_ref, scw_ref, scc_ref,
                  tabf_ref, tabx_ref, sc_ref):
    f = feats_ref[...]
    x1 = jnp.dot(f, u1w_ref[...], preferred_element_type=jnp.float32)
    c = u1c_ref[...]
    tabf_ref[...] = _lnorm(x1 + c[0:1], c[1:2], c[2:3])
    pad = jnp.zeros((f.shape[0], D_X - 3), jnp.float32)
    tabx_ref[...] = jnp.concatenate([xyz_ref[...], pad], axis=1)
    s1 = jnp.dot(f, scw_ref[...], preferred_element_type=jnp.float32)
    cs = scc_ref[...]
    sc_ref[...] = _lnorm(s1 + cs[0:1], cs[1:2], cs[2:3])


def _stage_a(feats, xyz, u1w, u1c, scw, scc):
    n_blk = N // PB_A
    return pl.pallas_call(
        _stage_a_body,
        grid=(n_blk,),
        in_specs=[
            pl.BlockSpec((PB_A, IN_CH), lambda i: (i, 0)),
            pl.BlockSpec((PB_A, 3), lambda i: (i, 0)),
            pl.BlockSpec((IN_CH, 64), lambda i: (0, 0)),
            pl.BlockSpec((3, 64), lambda i: (0, 0)),
            pl.BlockSpec((IN_CH, OUT_CH), lambda i: (0, 0)),
            pl.BlockSpec((3, OUT_CH), lambda i: (0, 0)),
        ],
        out_specs=[
            pl.BlockSpec((PB_A, D_F), lambda i: (i, 0)),
            pl.BlockSpec((PB_A, D_X), lambda i: (i, 0)),
            pl.BlockSpec((PB_A, OUT_CH), lambda i: (i, 0)),
        ],
        out_shape=[
            jax.ShapeDtypeStruct((N, D_F), jnp.float32),
            jax.ShapeDtypeStruct((N, D_X), jnp.float32),
            jax.ShapeDtypeStruct((N, OUT_CH), jnp.float32),
        ],
    )(feats, xyz, u1w, u1c, scw, scc)


def _sc_gather(tabf, tabx, idx_flat, idx_ctr):
    info = plsc.get_sparse_core_info()
    nc, ns = info.num_cores, info.num_subcores
    nw = nc * ns
    e_tot = idx_flat.shape[0]
    b_per_w = e_tot // nw
    n_chunks = b_per_w // CB
    mesh = plsc.VectorSubcoreMesh(core_axis_name="c", subcore_axis_name="s")

    @functools.partial(
        pl.kernel,
        mesh=mesh,
        out_type=[
            jax.ShapeDtypeStruct((e_tot, D_F), jnp.float32),
            jax.ShapeDtypeStruct((e_tot, D_X), jnp.float32),
        ],
        compiler_params=pltpu.CompilerParams(use_tc_tiling_on_sc=False),
        scratch_types=[
            pltpu.VMEM((CB,), jnp.int32),
            pltpu.VMEM((CB,), jnp.int32),
            pltpu.VMEM((CB, D_F), jnp.float32),
            pltpu.VMEM((CB, D_X), jnp.float32),
            pltpu.VMEM((CB, D_X), jnp.float32),
            pltpu.SemaphoreType.DMA,
            pltpu.SemaphoreType.DMA,
            pltpu.SemaphoreType.DMA,
        ],
    )
    def gather_kernel(tabf_hbm, tabx_hbm, idx_hbm, idx2_hbm, outf_hbm,
                      loc_hbm, idx_v, idx2_v, rowsf_v, rowsx_v, ctr_v,
                      semf, semx, semc):
        wid = lax.axis_index("s") * nc + lax.axis_index("c")
        base = wid * b_per_w

        def body(i, carry):
            off = base + i * CB
            pltpu.sync_copy(idx_hbm.at[pl.ds(off, CB)], idx_v)
            pltpu.sync_copy(idx2_hbm.at[pl.ds(off, CB)], idx2_v)
            cpf = pltpu.async_copy(tabf_hbm.at[idx_v], rowsf_v, semf)
            cpx = pltpu.async_copy(tabx_hbm.at[idx_v], rowsx_v, semx)
            cpc = pltpu.async_copy(tabx_hbm.at[idx2_v], ctr_v, semc)
            cpf.wait()
            cpx.wait()
            cpc.wait()

            def sub_body(j, c2):
                rowsx_v[j] = rowsx_v[j] - ctr_v[j]
                return c2

            lax.fori_loop(0, CB, sub_body, 0)
            pltpu.sync_copy(rowsf_v, outf_hbm.at[pl.ds(off, CB)])
            pltpu.sync_copy(rowsx_v, loc_hbm.at[pl.ds(off, CB)])
            return carry

        lax.fori_loop(0, n_chunks, body, 0)

    return gather_kernel(tabf, tabx, idx_flat, idx_ctr)


GP = 8
GE = GP * K


def _stage_c_body(gathf_ref, loc_ref, sc_ref, wpe0_ref, pe0c_ref,
                  pew1_ref, pe1c_ref, wwn0_ref, wn0c_ref, wn1_ref, wn1c_ref,
                  wn2_ref, wn2c_ref, a3_ref, u2c_ref, mask_ref, out_ref,
                  gs_ref):
    n_grp = PB_C // GP
    fg = gathf_ref[...]
    loc = loc_ref[...]

    hpe = jnp.dot(loc, wpe0_ref[...], preferred_element_type=jnp.float32)
    h = _lrelu(_ln_aug(hpe, pe0c_ref[...], 64))
    pea = jnp.dot(h, pew1_ref[...], preferred_element_type=jnp.float32)
    pe = _ln_aug(pea, pe1c_ref[...], 32)

    hwn = jnp.dot(loc, wwn0_ref[...], preferred_element_type=jnp.float32)
    w0 = _lrelu(_ln_aug(hwn, wn0c_ref[...], 16))
    w1a = jnp.dot(w0, wn1_ref[...], preferred_element_type=jnp.float32)
    w1 = _lrelu(_ln_aug(w1a, wn1c_ref[...], 16))
    wtsa = jnp.dot(w1, wn2_ref[...], preferred_element_type=jnp.float32)
    wts = _ln_aug(wtsa, wn2c_ref[...], 16)

    feat = jnp.concatenate([fg, pe], axis=1)
    mask = mask_ref[...]
    for g in range(n_grp):
        wg = wts[g * GE:(g + 1) * GE]
        wbd = jnp.tile(wg, (1, GP)) * mask
        fgk = feat[g * GE:(g + 1) * GE]
        gd = lax.dot_general(wbd, fgk, (((0,), (0,)), ((), ())),
                             preferred_element_type=jnp.float32)
        gs_ref[g] = gd.reshape(GP, 16, 96)

    o = jnp.zeros((PB_C, OUT_CH), jnp.float32)
    for w in range(16):
        gw = gs_ref[:, :, w, :].reshape(PB_C, 96)
        o = o + jnp.dot(gw, a3_ref[w], preferred_element_type=jnp.float32)
    u2c = u2c_ref[...]
    o = _lnorm(o + u2c[0:1], u2c[1:2], u2c[2:3])
    out_ref[...] = _lrelu(o + sc_ref[...])


def _stage_c(gathf, loc, sc, wpe0, pe0c, pew1, pe1c, wwn0, wn0c, wn1, wn1c,
             wn2, wn2c, a3, u2c, mask):
    n_blk = N // PB_C
    e_blk = PB_C * K
    full = lambda shape: pl.BlockSpec(shape, lambda i: tuple(0 for _ in shape))
    return pl.pallas_call(
        _stage_c_body,
        grid=(n_blk,),
        in_specs=[
            pl.BlockSpec((e_blk, D_F), lambda i: (i, 0)),
            pl.BlockSpec((e_blk, D_X), lambda i: (i, 0)),
            pl.BlockSpec((PB_C, OUT_CH), lambda i: (i, 0)),
            full((D_X, 65)),
            full((4, 64)),
            full((64, 33)),
            full((4, 32)),
            full((D_X, 17)),
            full((4, 16)),
            full((16, 17)),
            full((4, 16)),
            full((16, 17)),
            full((4, 16)),
            full((16, 96, OUT_CH)),
            full((3, OUT_CH)),
            full((GE, GP * 16)),
        ],
        out_specs=pl.BlockSpec((PB_C, OUT_CH), lambda i: (i, 0)),
        out_shape=jax.ShapeDtypeStruct((N, OUT_CH), jnp.float32),
        scratch_shapes=[
            pltpu.VMEM((PB_C // GP, GP, 16, 96), jnp.float32),
        ],
    )(gathf, loc, sc, wpe0, pe0c, pew1, pe1c, wwn0, wn0c, wn1, wn1c,
      wn2, wn2c, a3, u2c, mask)


def kernel(dense_xyz, dense_feats, nei_inds, dense_xyz_norm, params):
    p = params
    xyz = dense_xyz[0]
    feats = dense_feats[0]
    idx = nei_inds[0].reshape(-1).astype(jnp.int32)

    u1c = jnp.stack([p['u1_b'], p['u1_g'], p['u1_be']])
    scc = jnp.stack([p['sc_b'], p['sc_g'], p['sc_be']])
    tabf, tabx, sc = _stage_a(feats, xyz, p['u1_W'], u1c, p['sc_W'], scc)

    idx_ctr = (jnp.arange(N * K, dtype=jnp.int32) // K).astype(jnp.int32)
    gathf, loc = _sc_gather(tabf, tabx, idx, idx_ctr)

    def aug(w):
        return jnp.concatenate([w, jnp.sum(w, axis=1, keepdims=True)], axis=1)

    def cpack(b, g, be):
        return jnp.stack([b, g, be, jnp.full_like(b, jnp.mean(b))])

    wpe0 = aug(jnp.zeros((D_X, 64), jnp.float32).at[:3].set(p['pe_W0']))
    wwn0 = aug(jnp.zeros((D_X, 16), jnp.float32).at[:3].set(p['wn_W0']))
    pe0c = cpack(p['pe_b0'], p['pe_g0'], p['pe_be0'])
    pew1 = aug(p['pe_W1'])
    pe1c = cpack(p['pe_b1'], p['pe_g1'], p['pe_be1'])
    wn0c = cpack(p['wn_b0'], p['wn_g0'], p['wn_be0'])
    wn1 = aug(p['wn_W1'])
    wn1c = cpack(p['wn_b1'], p['wn_g1'], p['wn_be1'])
    wn2 = aug(p['wn_W2'])
    wn2c = cpack(p['wn_b2'], p['wn_g2'], p['wn_be2'])
    a3 = p['u2_W'].reshape(96, 16, OUT_CH).transpose(1, 0, 2)
    u2c = jnp.stack([p['u2_b'], p['u2_g'], p['u2_be']])
    mask = jnp.kron(jnp.eye(GP, dtype=jnp.float32),
                    jnp.ones((K, 16), jnp.float32))

    out = _stage_c(gathf, loc, sc, wpe0, pe0c, pew1, pe1c,
                   wwn0, wn0c, wn1, wn1c, wn2, wn2c, a3, u2c, mask)

    return out[None], loc.reshape(1, N, K, D_X)[..., :3]


# speedup vs baseline: 1.1190x; 1.1190x over previous
import functools

import jax
import jax.numpy as jnp
from jax import lax
from jax.experimental import pallas as pl
from jax.experimental.pallas import tpu as pltpu
from jax.experimental.pallas import tpu_sc as plsc

N = 50000
K = 16
IN_CH = 128
OUT_CH = 256
D_F = 64
D_X = 16
PB_A = 2000
PB_C = 400
CB = 1000
GP = 8
GE = GP * K


def _lnorm(x, g, b, eps=1e-5):
    n_inv = 1.0 / x.shape[-1]
    mu = jnp.sum(x, axis=-1, keepdims=True) * n_inv
    var = jnp.sum(x * x, axis=-1, keepdims=True) * n_inv - mu * mu
    r = lax.rsqrt(var + eps)
    return (x - mu) * r * g + b


def _lrelu(x):
    return jnp.where(x >= 0, x, 0.1 * x)


def _stage_a_body(feats_ref, xyz_ref, u1w_ref, u1c_ref, scw_ref, scc_ref,
                  tabf_ref, tabx_ref, sc_ref):
    f = feats_ref[...]
    x1 = jnp.dot(f, u1w_ref[...], preferred_element_type=jnp.float32)
    c = u1c_ref[...]
    tabf_ref[...] = _lnorm(x1 + c[0:1], c[1:2], c[2:3])
    pad = jnp.zeros((f.shape[0], D_X - 3), jnp.float32)
    tabx_ref[...] = jnp.concatenate([xyz_ref[...], pad], axis=1)
    s1 = jnp.dot(f, scw_ref[...], preferred_element_type=jnp.float32)
    cs = scc_ref[...]
    sc_ref[...] = _lnorm(s1 + cs[0:1], cs[1:2], cs[2:3])


def _stage_a(feats, xyz, u1w, u1c, scw, scc):
    n_blk = N // PB_A
    return pl.pallas_call(
        _stage_a_body,
        grid=(n_blk,),
        in_specs=[
            pl.BlockSpec((PB_A, IN_CH), lambda i: (i, 0)),
            pl.BlockSpec((PB_A, 3), lambda i: (i, 0)),
            pl.BlockSpec((IN_CH, 64), lambda i: (0, 0)),
            pl.BlockSpec((3, 64), lambda i: (0, 0)),
            pl.BlockSpec((IN_CH, OUT_CH), lambda i: (0, 0)),
            pl.BlockSpec((3, OUT_CH), lambda i: (0, 0)),
        ],
        out_specs=[
            pl.BlockSpec((PB_A, D_F), lambda i: (i, 0)),
            pl.BlockSpec((PB_A, D_X), lambda i: (i, 0)),
            pl.BlockSpec((PB_A, OUT_CH), lambda i: (i, 0)),
        ],
        out_shape=[
            jax.ShapeDtypeStruct((N, D_F), jnp.float32),
            jax.ShapeDtypeStruct((N, D_X), jnp.float32),
            jax.ShapeDtypeStruct((N, OUT_CH), jnp.float32),
        ],
    )(feats, xyz, u1w, u1c, scw, scc)


def _sc_gather(tabf, tabx, idx_flat):
    info = plsc.get_sparse_core_info()
    nc, ns = info.num_cores, info.num_subcores
    nw = nc * ns
    e_tot = idx_flat.shape[0]
    b_per_w = e_tot // nw
    n_chunks = b_per_w // CB
    mesh = plsc.VectorSubcoreMesh(core_axis_name="c", subcore_axis_name="s")

    @functools.partial(
        pl.kernel,
        mesh=mesh,
        out_type=[
            jax.ShapeDtypeStruct((e_tot, D_F), jnp.float32),
            jax.ShapeDtypeStruct((e_tot, D_X), jnp.float32),
        ],
        compiler_params=pltpu.CompilerParams(use_tc_tiling_on_sc=False),
        scratch_types=[
            pltpu.VMEM((CB,), jnp.int32),
            pltpu.VMEM((CB, D_F), jnp.float32),
            pltpu.VMEM((CB, D_X), jnp.float32),
            pltpu.SemaphoreType.DMA,
            pltpu.SemaphoreType.DMA,
        ],
    )
    def gather_kernel(tabf_hbm, tabx_hbm, idx_hbm, outf_hbm, outx_hbm,
                      idx_v, rowsf_v, rowsx_v, semf, semx):
        wid = lax.axis_index("s") * nc + lax.axis_index("c")
        base = wid * b_per_w

        def body(i, carry):
            off = base + i * CB
            pltpu.sync_copy(idx_hbm.at[pl.ds(off, CB)], idx_v)
            cpf = pltpu.async_copy(tabf_hbm.at[idx_v], rowsf_v, semf)
            cpx = pltpu.async_copy(tabx_hbm.at[idx_v], rowsx_v, semx)
            cpf.wait()
            cpx.wait()
            pltpu.sync_copy(rowsf_v, outf_hbm.at[pl.ds(off, CB)])
            pltpu.sync_copy(rowsx_v, outx_hbm.at[pl.ds(off, CB)])
            return carry

        lax.fori_loop(0, n_chunks, body, 0)

    return gather_kernel(tabf, tabx, idx_flat)


def _stage_c_body(gathf_ref, gathx_ref, xyz_ref, sc_ref, w03_ref, pe0c_ref,
                  pew1_ref, pe1c_ref, wn0c_ref, wn1_ref, wn2_ref, a3_ref,
                  u2c_ref, mask_ref, out_ref, loc_ref, gs_ref):
    n_grp = PB_C // GP
    fg = gathf_ref[...]
    gx3 = gathx_ref[...].reshape(PB_C, K, D_X)
    lx3 = gx3 - xyz_ref[...][:, None, :]
    loc = lx3.reshape(PB_C * K, D_X)
    loc_ref[...] = loc

    hw = jnp.dot(loc, w03_ref[...], preferred_element_type=jnp.float32)
    pe0c = pe0c_ref[...]
    h = _lrelu(_lnorm(hw[:, :64] + pe0c[0:1], pe0c[1:2], pe0c[2:3]))
    pe1c = pe1c_ref[...]
    pe = jnp.dot(h, pew1_ref[...], preferred_element_type=jnp.float32) + pe1c[0:1]
    pe = _lnorm(pe, pe1c[1:2], pe1c[2:3])

    wn0c = wn0c_ref[...]
    w0 = _lrelu(_lnorm(hw[:, 64:80] + wn0c[0:1], wn0c[1:2], wn0c[2:3]))
    wn1 = wn1_ref[...]
    w1 = jnp.dot(w0, wn1[:16], preferred_element_type=jnp.float32) + wn1[16:17]
    w1 = _lrelu(_lnorm(w1, wn1[17:18], wn1[18:19]))
    wn2 = wn2_ref[...]
    wts = jnp.dot(w1, wn2[:16], preferred_element_type=jnp.float32) + wn2[16:17]
    wts = _lnorm(wts, wn2[17:18], wn2[18:19])

    feat = jnp.concatenate([fg, pe], axis=1)
    mask = mask_ref[...]
    for g in range(n_grp):
        wg = wts[g * GE:(g + 1) * GE]
        wbd = jnp.tile(wg, (1, GP)) * mask
        fgk = feat[g * GE:(g + 1) * GE]
        gd = lax.dot_general(wbd, fgk, (((0,), (0,)), ((), ())),
                             preferred_element_type=jnp.float32)
        gs_ref[g] = gd.reshape(GP, 16, 96)

    o = jnp.zeros((PB_C, OUT_CH), jnp.float32)
    for w in range(16):
        gw = gs_ref[:, :, w, :].reshape(PB_C, 96)
        o = o + jnp.dot(gw, a3_ref[w], preferred_element_type=jnp.float32)
    u2c = u2c_ref[...]
    o = _lnorm(o + u2c[0:1], u2c[1:2], u2c[2:3])
    out_ref[...] = _lrelu(o + sc_ref[...])


def _stage_c(gathf, gathx, xyz, sc, w03, pe0c, pew1, pe1c, wn0c, wn1, wn2,
             a3, u2c, mask):
    n_blk = N // PB_C
    e_blk = PB_C * K
    full = lambda shape: pl.BlockSpec(shape, lambda i: tuple(0 for _ in shape))
    return pl.pallas_call(
        _stage_c_body,
        grid=(n_blk,),
        in_specs=[
            pl.BlockSpec((e_blk, D_F), lambda i: (i, 0)),
            pl.BlockSpec((e_blk, D_X), lambda i: (i, 0)),
            pl.BlockSpec((PB_C, D_X), lambda i: (i, 0)),
            pl.BlockSpec((PB_C, OUT_CH), lambda i: (i, 0)),
            full((D_X, 80)),
            full((3, 64)),
            full((64, 32)),
            full((3, 32)),
            full((3, 16)),
            full((19, 16)),
            full((19, 16)),
            full((16, 96, OUT_CH)),
            full((3, OUT_CH)),
            full((GE, GP * 16)),
        ],
        out_specs=[
            pl.BlockSpec((PB_C, OUT_CH), lambda i: (i, 0)),
            pl.BlockSpec((e_blk, D_X), lambda i: (i, 0)),
        ],
        out_shape=[
            jax.ShapeDtypeStruct((N, OUT_CH), jnp.float32),
            jax.ShapeDtypeStruct((N * K, D_X), jnp.float32),
        ],
        scratch_shapes=[
            pltpu.VMEM((PB_C // GP, GP, 16, 96), jnp.float32),
        ],
    )(gathf, gathx, xyz, sc, w03, pe0c, pew1, pe1c, wn0c, wn1, wn2, a3,
      u2c, mask)


def kernel(dense_xyz, dense_feats, nei_inds, dense_xyz_norm, params):
    p = params
    xyz = dense_xyz[0]
    feats = dense_feats[0]
    idx = nei_inds[0].reshape(-1).astype(jnp.int32)

    u1c = jnp.stack([p['u1_b'], p['u1_g'], p['u1_be']])
    scc = jnp.stack([p['sc_b'], p['sc_g'], p['sc_be']])
    tabf, tabx, sc = _stage_a(feats, xyz, p['u1_W'], u1c, p['sc_W'], scc)

    gathf, gathx = _sc_gather(tabf, tabx, idx)

    w03 = jnp.zeros((D_X, 80), jnp.float32).at[:3].set(
        jnp.concatenate([p['pe_W0'], p['wn_W0']], axis=1))
    pe0c = jnp.stack([p['pe_b0'], p['pe_g0'], p['pe_be0']])
    pe1c = jnp.stack([p['pe_b1'], p['pe_g1'], p['pe_be1']])
    wn0c = jnp.stack([p['wn_b0'], p['wn_g0'], p['wn_be0']])
    wn1 = jnp.concatenate(
        [p['wn_W1'], p['wn_b1'][None], p['wn_g1'][None], p['wn_be1'][None]], 0)
    wn2 = jnp.concatenate(
        [p['wn_W2'], p['wn_b2'][None], p['wn_g2'][None], p['wn_be2'][None]], 0)
    a3 = p['u2_W'].reshape(96, 16, OUT_CH).transpose(1, 0, 2)
    u2c = jnp.stack([p['u2_b'], p['u2_g'], p['u2_be']])
    mask = jnp.kron(jnp.eye(GP, dtype=jnp.float32),
                    jnp.ones((K, 16), jnp.float32))

    out, loc = _stage_c(gathf, gathx, tabx, sc, w03, pe0c, p['pe_W1'], pe1c,
                        wn0c, wn1, wn2, a3, u2c, mask)

    return out[None], loc.reshape(1, N, K, D_X)[..., :3]
